# Initial kernel scaffold; baseline (speedup 1.0000x reference)
import jax
import jax.numpy as jnp
from jax.experimental import pallas as pl


def kernel(x, edge_index, edge_type, batch, W1, root1, b1, W2, root2, b2, W3, root3, b3, fcW, fcb):
    raise NotImplementedError("write your pallas kernel here")


# speedup vs baseline: 6.9152x; 6.9152x over previous
import functools

import jax
import jax.numpy as jnp
from jax import lax
from jax.experimental import pallas as pl
from jax.experimental.pallas import tpu as pltpu
from jax.experimental.pallas import tpu_sc as plsc

N = 10000
E = 320000
R = 8
EPAD = 327680
SUB = 128
NSUB = 160
GRP = 8
ACC_ROWS = 80128
ZROWS = 626
IGRP = 32
DUMP = R * N
BN = 512
NPAD = 10240


def _sc_aggregate(k, xT, zeros_init, src16, sidx16):
    k_half = k // 2
    mesh = plsc.VectorSubcoreMesh(core_axis_name="c", subcore_axis_name="s")

    @functools.partial(
        pl.kernel,
        mesh=mesh,
        out_type=jax.ShapeDtypeStruct((k, R * N, 16), jnp.float32),
        compiler_params=pltpu.CompilerParams(use_tc_tiling_on_sc=False),
        scratch_types=[
            pltpu.VMEM((IGRP, SUB), jnp.int32),
            pltpu.VMEM((IGRP, SUB), jnp.int32),
            pltpu.VMEM((GRP * SUB, 16), jnp.float32),
            pltpu.VMEM((ZROWS, 16), jnp.float32),
            pltpu.VMEM_SHARED((ACC_ROWS, 16), jnp.float32),
            pltpu.SemaphoreType.DMA,
        ],
    )
    def body(xT_hbm, zer_hbm, src_hbm, sidx_hbm, out_hbm,
             src_t, sidx_t, rows_t, zbuf, acc, gsem):
        core = lax.axis_index("c")
        s = lax.axis_index("s")
        pltpu.sync_copy(zer_hbm, zbuf)

        def chunk_body(cc, carry):
            c = core * k_half + cc
            for z in range(8):
                pltpu.sync_copy(zbuf, acc.at[pl.ds(s * 5008 + z * ZROWS, ZROWS)])
            plsc.subcore_barrier()

            def igrp_body(m, carry1):
                pltpu.sync_copy(src_hbm.at[s].at[pl.ds(m * IGRP, IGRP)], src_t)
                pltpu.sync_copy(sidx_hbm.at[s].at[pl.ds(m * IGRP, IGRP)],
                                sidx_t)

                def grp_body(g, carry2):
                    handles = []
                    for j in range(GRP):
                        row = g * GRP + j
                        handles.append(pltpu.async_copy(
                            xT_hbm.at[c].at[src_t.at[row]],
                            rows_t.at[pl.ds(j * SUB, SUB)], gsem))
                    for h in handles:
                        h.wait()
                    for j in range(GRP):
                        row = g * GRP + j
                        pltpu.sync_copy(rows_t.at[pl.ds(j * SUB, SUB)],
                                        acc.at[sidx_t.at[row]], add=True)
                    return carry2

                lax.fori_loop(0, IGRP // GRP, grp_body, 0)
                return carry1

            lax.fori_loop(0, NSUB // IGRP, igrp_body, 0)
            plsc.subcore_barrier()
            pltpu.sync_copy(acc.at[pl.ds(s * 5000, 5000)],
                            out_hbm.at[c].at[pl.ds(s * 5000, 5000)])
            plsc.subcore_barrier()
            return carry

        lax.fori_loop(0, k_half, chunk_body, 0)

    return body(xT, zeros_init, src16, sidx16)


def _sc_count(sidx32, zeros_init, ones_init):
    mesh = plsc.VectorSubcoreMesh(core_axis_name="c", subcore_axis_name="s")

    @functools.partial(
        pl.kernel,
        mesh=mesh,
        out_type=jax.ShapeDtypeStruct((2, R * N, 16), jnp.float32),
        compiler_params=pltpu.CompilerParams(use_tc_tiling_on_sc=False),
        scratch_types=[
            pltpu.VMEM((80, SUB), jnp.int32),
            pltpu.VMEM((SUB, 16), jnp.float32),
            pltpu.VMEM((ZROWS, 16), jnp.float32),
            pltpu.VMEM_SHARED((ACC_ROWS, 16), jnp.float32),
        ],
    )
    def body(sidx_hbm, zer_hbm, one_hbm, out_hbm, sidx_t, ones_t, zbuf, acc):
        core = lax.axis_index("c")
        s = lax.axis_index("s")
        wid = s * 2 + core
        pltpu.sync_copy(sidx_hbm.at[wid], sidx_t)
        pltpu.sync_copy(one_hbm, ones_t)
        pltpu.sync_copy(zer_hbm, zbuf)
        for z in range(8):
            pltpu.sync_copy(zbuf, acc.at[pl.ds(s * 5008 + z * ZROWS, ZROWS)])
        plsc.subcore_barrier()

        def sub_body(g, carry):
            pltpu.sync_copy(ones_t, acc.at[sidx_t.at[g]], add=True)
            return carry

        lax.fori_loop(0, 80, sub_body, 0)
        plsc.subcore_barrier()
        pltpu.sync_copy(acc.at[pl.ds(s * 5000, 5000)],
                        out_hbm.at[core].at[pl.ds(s * 5000, 5000)])

    return body(sidx32, zeros_init, ones_init)


def _tc_layer(x, agg, cnt2, w, root, b, dout):
    d = x.shape[1]

    def body(x_ref, agg_ref, cnt_ref, w_ref, root_ref, b_ref, out_ref):
        xb = x_ref[...]
        cnt = cnt_ref[0] + cnt_ref[1]
        inv = 1.0 / jnp.maximum(cnt, 1.0)
        h = jnp.dot(xb, root_ref[...], preferred_element_type=jnp.float32)
        h = h + b_ref[...]
        for r in range(R):
            h = h + jnp.dot(agg_ref[r] * inv[r][:, None], w_ref[r],
                            preferred_element_type=jnp.float32)
        out_ref[...] = jnp.maximum(h, 0.0)

    return pl.pallas_call(
        body,
        grid=(NPAD // BN,),
        in_specs=[
            pl.BlockSpec((BN, d), lambda i: (i, 0)),
            pl.BlockSpec((R, BN, d), lambda i: (0, i, 0)),
            pl.BlockSpec((2, R, BN), lambda i: (0, 0, i)),
            pl.BlockSpec((R, d, dout), lambda i: (0, 0, 0)),
            pl.BlockSpec((d, dout), lambda i: (0, 0)),
            pl.BlockSpec((1, dout), lambda i: (0, 0)),
        ],
        out_specs=pl.BlockSpec((BN, dout), lambda i: (i, 0)),
        out_shape=jax.ShapeDtypeStruct((NPAD, dout), jnp.float32),
    )(x, agg, cnt2, w, root, b)


def _tc_layer3_pool(x, agg, cnt2, w, root, b, fcwE, batch3):
    d = x.shape[1]

    def body(x_ref, agg_ref, cnt_ref, w_ref, root_ref, b_ref, fcw_ref,
             bat_ref, out_ref):
        i = pl.program_id(0)
        xb = x_ref[...]
        cnt = cnt_ref[0] + cnt_ref[1]
        inv = 1.0 / jnp.maximum(cnt, 1.0)
        h = jnp.dot(xb, root_ref[...], preferred_element_type=jnp.float32)
        h = h + b_ref[...]
        for r in range(R):
            h = h + jnp.dot(agg_ref[r] * inv[r][:, None], w_ref[r],
                            preferred_element_type=jnp.float32)
        h = jnp.maximum(h, 0.0)
        zc = jnp.dot(h, fcw_ref[...], preferred_element_type=jnp.float32)
        ones_col = (lax.broadcasted_iota(jnp.int32, (BN, 128), 1) == 1)
        zc = zc + ones_col.astype(jnp.float32)
        bat = bat_ref[...].reshape(BN)
        oh = (bat[None, :] == lax.broadcasted_iota(jnp.int32, (64, BN), 0))
        contrib = jnp.dot(oh.astype(jnp.float32), zc,
                          preferred_element_type=jnp.float32)

        @pl.when(i == 0)
        def _():
            out_ref[...] = contrib

        @pl.when(i > 0)
        def _():
            out_ref[...] = out_ref[...] + contrib

    return pl.pallas_call(
        body,
        grid=(NPAD // BN,),
        in_specs=[
            pl.BlockSpec((BN, d), lambda i: (i, 0)),
            pl.BlockSpec((R, BN, d), lambda i: (0, i, 0)),
            pl.BlockSpec((2, R, BN), lambda i: (0, 0, i)),
            pl.BlockSpec((R, d, 32), lambda i: (0, 0, 0)),
            pl.BlockSpec((d, 32), lambda i: (0, 0)),
            pl.BlockSpec((1, 32), lambda i: (0, 0)),
            pl.BlockSpec((32, 128), lambda i: (0, 0)),
            pl.BlockSpec((1, 1, BN), lambda i: (i, 0, 0)),
        ],
        out_specs=pl.BlockSpec((64, 128), lambda i: (0, 0)),
        out_shape=jax.ShapeDtypeStruct((64, 128), jnp.float32),
    )(x, agg, cnt2, w, root, b, fcwE, batch3)


def _tc_finish(pooled2, fcb2):
    def body(p_ref, fcb_ref, out_ref):
        a = p_ref[...]
        v = a[:, 0:1] / jnp.maximum(a[:, 1:2], 1.0) + fcb_ref[0, 0]
        out_ref[...] = jnp.broadcast_to(v, (64, 128))

    return pl.pallas_call(
        body,
        in_specs=[pl.BlockSpec((64, 128), lambda: (0, 0)),
                  pl.BlockSpec((1, 1), lambda: (0, 0))],
        out_specs=pl.BlockSpec((64, 128), lambda: (0, 0)),
        out_shape=jax.ShapeDtypeStruct((64, 128), jnp.float32),
    )(pooled2, fcb2)


def _pad_nodes(a):
    return jnp.pad(a, ((0, NPAD - N), (0, 0)))


def _agg_to_dense(raw, k, d):
    t = raw.reshape(k, R, N, 16).transpose(1, 2, 0, 3).reshape(R, N, d)
    return jnp.pad(t, ((0, 0), (0, NPAD - N), (0, 0)))


def kernel(x, edge_index, edge_type, batch, W1, root1, b1, W2, root2, b2,
           W3, root3, b3, fcW, fcb):
    src = edge_index[0].astype(jnp.int32)
    dst = edge_index[1].astype(jnp.int32)
    et = edge_type.astype(jnp.int32)

    pad = EPAD - E
    src_p = jnp.concatenate([src, jnp.zeros((pad,), jnp.int32)])
    sidx = et * N + dst
    sidx_p = jnp.concatenate([sidx, jnp.full((pad,), DUMP, jnp.int32)])
    src16 = src_p.reshape(16, NSUB, SUB)
    sidx16 = sidx_p.reshape(16, NSUB, SUB)
    sidx32 = sidx_p.reshape(32, 80, SUB)
    zeros_init = jnp.zeros((ZROWS, 16), jnp.float32)
    ones_init = jnp.ones((SUB, 16), jnp.float32)

    cnt_raw = _sc_count(sidx32, zeros_init, ones_init)
    cnt2 = cnt_raw[:, :, 0].reshape(2, R, N)
    cnt2 = jnp.pad(cnt2, ((0, 0), (0, 0), (0, NPAD - N)))

    xT1 = x.reshape(N, 8, 16).transpose(1, 0, 2)
    agg1 = _sc_aggregate(8, xT1, zeros_init, src16, sidx16)
    h1 = _tc_layer(_pad_nodes(x), _agg_to_dense(agg1, 8, 128), cnt2,
                   W1, root1, b1.reshape(1, 256), 256)

    xT2 = h1[:N].reshape(N, 16, 16).transpose(1, 0, 2)
    agg2 = _sc_aggregate(16, xT2, zeros_init, src16, sidx16)
    h2 = _tc_layer(h1, _agg_to_dense(agg2, 16, 256), cnt2,
                   W2, root2, b2.reshape(1, 256), 256)

    xT3 = h2[:N].reshape(N, 16, 16).transpose(1, 0, 2)
    agg3 = _sc_aggregate(16, xT3, zeros_init, src16, sidx16)
    fcwE = jnp.pad(fcW, ((0, 0), (0, 127)))
    batch3 = jnp.concatenate(
        [batch.astype(jnp.int32), jnp.full((NPAD - N,), 64, jnp.int32)]
    ).reshape(NPAD // BN, 1, BN)
    pooled2 = _tc_layer3_pool(h2, _agg_to_dense(agg3, 16, 256), cnt2,
                              W3, root3, b3.reshape(1, 32), fcwE, batch3)

    fin = _tc_finish(pooled2, fcb.reshape(1, 1))
    return fin[:, :1]


# speedup vs baseline: 7.4699x; 1.0802x over previous
import functools

import jax
import jax.numpy as jnp
from jax import lax
from jax.experimental import pallas as pl
from jax.experimental.pallas import tpu as pltpu
from jax.experimental.pallas import tpu_sc as plsc

N = 10000
E = 320000
R = 8
EPAD = 327680
SUB = 128
NSUB = 160
GRP = 8
ACC_ROWS = 80128
ZROWS = 313
IGRP = 32
DUMP = R * N
BN = 512
NPAD = 10240


def _sc_aggregate(k, xT, zeros_init, src16, sidx16):
    k_half = k // 2
    mesh = plsc.VectorSubcoreMesh(core_axis_name="c", subcore_axis_name="s")

    @functools.partial(
        pl.kernel,
        mesh=mesh,
        out_type=jax.ShapeDtypeStruct((k, R * N, 16), jnp.float32),
        compiler_params=pltpu.CompilerParams(use_tc_tiling_on_sc=False),
        scratch_types=[
            pltpu.VMEM((IGRP, SUB), jnp.int32),
            pltpu.VMEM((IGRP, SUB), jnp.int32),
            pltpu.VMEM((2 * GRP * SUB, 16), jnp.float32),
            pltpu.VMEM((ZROWS, 16), jnp.float32),
            pltpu.VMEM_SHARED((ACC_ROWS, 16), jnp.float32),
            pltpu.SemaphoreType.DMA,
            pltpu.SemaphoreType.DMA,
        ],
    )
    def body(xT_hbm, zer_hbm, src_hbm, sidx_hbm, out_hbm,
             src_t, sidx_t, rows_t, zbuf, acc, gsem, ssem):
        core = lax.axis_index("c")
        s = lax.axis_index("s")
        pltpu.sync_copy(zer_hbm, zbuf)
        ngrp = IGRP // GRP

        def fire_gathers(c, g, buf):
            return [pltpu.async_copy(
                xT_hbm.at[c].at[src_t.at[g * GRP + j]],
                rows_t.at[pl.ds(buf * GRP * SUB + j * SUB, SUB)], gsem)
                for j in range(GRP)]

        def chunk_body(cc, carry):
            c = core * k_half + cc
            for z in range(16):
                pltpu.sync_copy(zbuf, acc.at[pl.ds(s * 5008 + z * ZROWS, ZROWS)])
            plsc.subcore_barrier()

            def igrp_body(m, carry1):
                pltpu.sync_copy(src_hbm.at[s].at[pl.ds(m * IGRP, IGRP)], src_t)
                pltpu.sync_copy(sidx_hbm.at[s].at[pl.ds(m * IGRP, IGRP)],
                                sidx_t)
                gh = fire_gathers(c, 0, 0)
                for g in range(ngrp):
                    buf = g % 2
                    for h in gh:
                        h.wait()
                    if g + 1 < ngrp:
                        gh = fire_gathers(c, g + 1, 1 - buf)
                    sh = [pltpu.async_copy(
                        rows_t.at[pl.ds(buf * GRP * SUB + j * SUB, SUB)],
                        acc.at[sidx_t.at[g * GRP + j]], ssem, add=True)
                        for j in range(GRP)]
                    for h in sh:
                        h.wait()
                return carry1

            lax.fori_loop(0, NSUB // IGRP, igrp_body, 0)
            plsc.subcore_barrier()
            pltpu.sync_copy(acc.at[pl.ds(s * 5000, 5000)],
                            out_hbm.at[c].at[pl.ds(s * 5000, 5000)])
            plsc.subcore_barrier()
            return carry

        lax.fori_loop(0, k_half, chunk_body, 0)

    return body(xT, zeros_init, src16, sidx16)


def _sc_count(sidx32, zeros_init, ones_init):
    mesh = plsc.VectorSubcoreMesh(core_axis_name="c", subcore_axis_name="s")

    @functools.partial(
        pl.kernel,
        mesh=mesh,
        out_type=jax.ShapeDtypeStruct((2, R * N, 16), jnp.float32),
        compiler_params=pltpu.CompilerParams(use_tc_tiling_on_sc=False),
        scratch_types=[
            pltpu.VMEM((80, SUB), jnp.int32),
            pltpu.VMEM((SUB, 16), jnp.float32),
            pltpu.VMEM((ZROWS, 16), jnp.float32),
            pltpu.VMEM_SHARED((ACC_ROWS, 16), jnp.float32),
        ],
    )
    def body(sidx_hbm, zer_hbm, one_hbm, out_hbm, sidx_t, ones_t, zbuf, acc):
        core = lax.axis_index("c")
        s = lax.axis_index("s")
        wid = s * 2 + core
        pltpu.sync_copy(sidx_hbm.at[wid], sidx_t)
        pltpu.sync_copy(one_hbm, ones_t)
        pltpu.sync_copy(zer_hbm, zbuf)
        for z in range(16):
            pltpu.sync_copy(zbuf, acc.at[pl.ds(s * 5008 + z * ZROWS, ZROWS)])
        plsc.subcore_barrier()

        def sub_body(g, carry):
            pltpu.sync_copy(ones_t, acc.at[sidx_t.at[g]], add=True)
            return carry

        lax.fori_loop(0, 80, sub_body, 0)
        plsc.subcore_barrier()
        pltpu.sync_copy(acc.at[pl.ds(s * 5000, 5000)],
                        out_hbm.at[core].at[pl.ds(s * 5000, 5000)])

    return body(sidx32, zeros_init, ones_init)


def _tc_layer(x, agg, cnt2, w, root, b, dout):
    d = x.shape[1]

    def body(x_ref, agg_ref, cnt_ref, w_ref, root_ref, b_ref, out_ref):
        xb = x_ref[...]
        cnt = cnt_ref[0] + cnt_ref[1]
        inv = 1.0 / jnp.maximum(cnt, 1.0)
        h = jnp.dot(xb, root_ref[...], preferred_element_type=jnp.float32)
        h = h + b_ref[...]
        for r in range(R):
            h = h + jnp.dot(agg_ref[r] * inv[r][:, None], w_ref[r],
                            preferred_element_type=jnp.float32)
        out_ref[...] = jnp.maximum(h, 0.0)

    return pl.pallas_call(
        body,
        grid=(NPAD // BN,),
        in_specs=[
            pl.BlockSpec((BN, d), lambda i: (i, 0)),
            pl.BlockSpec((R, BN, d), lambda i: (0, i, 0)),
            pl.BlockSpec((2, R, BN), lambda i: (0, 0, i)),
            pl.BlockSpec((R, d, dout), lambda i: (0, 0, 0)),
            pl.BlockSpec((d, dout), lambda i: (0, 0)),
            pl.BlockSpec((1, dout), lambda i: (0, 0)),
        ],
        out_specs=pl.BlockSpec((BN, dout), lambda i: (i, 0)),
        out_shape=jax.ShapeDtypeStruct((NPAD, dout), jnp.float32),
    )(x, agg, cnt2, w, root, b)


def _tc_layer3_pool(x, agg, cnt2, w, root, b, fcwE, batch3):
    d = x.shape[1]

    def body(x_ref, agg_ref, cnt_ref, w_ref, root_ref, b_ref, fcw_ref,
             bat_ref, out_ref):
        i = pl.program_id(0)
        xb = x_ref[...]
        cnt = cnt_ref[0] + cnt_ref[1]
        inv = 1.0 / jnp.maximum(cnt, 1.0)
        h = jnp.dot(xb, root_ref[...], preferred_element_type=jnp.float32)
        h = h + b_ref[...]
        for r in range(R):
            h = h + jnp.dot(agg_ref[r] * inv[r][:, None], w_ref[r],
                            preferred_element_type=jnp.float32)
        h = jnp.maximum(h, 0.0)
        zc = jnp.dot(h, fcw_ref[...], preferred_element_type=jnp.float32)
        ones_col = (lax.broadcasted_iota(jnp.int32, (BN, 128), 1) == 1)
        zc = zc + ones_col.astype(jnp.float32)
        bat = bat_ref[...].reshape(BN)
        oh = (bat[None, :] == lax.broadcasted_iota(jnp.int32, (64, BN), 0))
        contrib = jnp.dot(oh.astype(jnp.float32), zc,
                          preferred_element_type=jnp.float32)

        @pl.when(i == 0)
        def _():
            out_ref[...] = contrib

        @pl.when(i > 0)
        def _():
            out_ref[...] = out_ref[...] + contrib

    return pl.pallas_call(
        body,
        grid=(NPAD // BN,),
        in_specs=[
            pl.BlockSpec((BN, d), lambda i: (i, 0)),
            pl.BlockSpec((R, BN, d), lambda i: (0, i, 0)),
            pl.BlockSpec((2, R, BN), lambda i: (0, 0, i)),
            pl.BlockSpec((R, d, 32), lambda i: (0, 0, 0)),
            pl.BlockSpec((d, 32), lambda i: (0, 0)),
            pl.BlockSpec((1, 32), lambda i: (0, 0)),
            pl.BlockSpec((32, 128), lambda i: (0, 0)),
            pl.BlockSpec((1, 1, BN), lambda i: (i, 0, 0)),
        ],
        out_specs=pl.BlockSpec((64, 128), lambda i: (0, 0)),
        out_shape=jax.ShapeDtypeStruct((64, 128), jnp.float32),
    )(x, agg, cnt2, w, root, b, fcwE, batch3)


def _tc_finish(pooled2, fcb2):
    def body(p_ref, fcb_ref, out_ref):
        a = p_ref[...]
        v = a[:, 0:1] / jnp.maximum(a[:, 1:2], 1.0) + fcb_ref[0, 0]
        out_ref[...] = jnp.broadcast_to(v, (64, 128))

    return pl.pallas_call(
        body,
        in_specs=[pl.BlockSpec((64, 128), lambda: (0, 0)),
                  pl.BlockSpec((1, 1), lambda: (0, 0))],
        out_specs=pl.BlockSpec((64, 128), lambda: (0, 0)),
        out_shape=jax.ShapeDtypeStruct((64, 128), jnp.float32),
    )(pooled2, fcb2)


def _pad_nodes(a):
    return jnp.pad(a, ((0, NPAD - N), (0, 0)))


def _agg_to_dense(raw, k, d):
    t = raw.reshape(k, R, N, 16).transpose(1, 2, 0, 3).reshape(R, N, d)
    return jnp.pad(t, ((0, 0), (0, NPAD - N), (0, 0)))


def kernel(x, edge_index, edge_type, batch, W1, root1, b1, W2, root2, b2,
           W3, root3, b3, fcW, fcb):
    src = edge_index[0].astype(jnp.int32)
    dst = edge_index[1].astype(jnp.int32)
    et = edge_type.astype(jnp.int32)

    pad = EPAD - E
    src_p = jnp.concatenate([src, jnp.zeros((pad,), jnp.int32)])
    sidx = et * N + dst
    sidx_p = jnp.concatenate([sidx, jnp.full((pad,), DUMP, jnp.int32)])
    src16 = src_p.reshape(16, NSUB, SUB)
    sidx16 = sidx_p.reshape(16, NSUB, SUB)
    sidx32 = sidx_p.reshape(32, 80, SUB)
    zeros_init = jnp.zeros((ZROWS, 16), jnp.float32)
    ones_init = jnp.ones((SUB, 16), jnp.float32)

    cnt_raw = _sc_count(sidx32, zeros_init, ones_init)
    cnt2 = cnt_raw[:, :, 0].reshape(2, R, N)
    cnt2 = jnp.pad(cnt2, ((0, 0), (0, 0), (0, NPAD - N)))

    xT1 = x.reshape(N, 8, 16).transpose(1, 0, 2)
    agg1 = _sc_aggregate(8, xT1, zeros_init, src16, sidx16)
    h1 = _tc_layer(_pad_nodes(x), _agg_to_dense(agg1, 8, 128), cnt2,
                   W1, root1, b1.reshape(1, 256), 256)

    xT2 = h1[:N].reshape(N, 16, 16).transpose(1, 0, 2)
    agg2 = _sc_aggregate(16, xT2, zeros_init, src16, sidx16)
    h2 = _tc_layer(h1, _agg_to_dense(agg2, 16, 256), cnt2,
                   W2, root2, b2.reshape(1, 256), 256)

    xT3 = h2[:N].reshape(N, 16, 16).transpose(1, 0, 2)
    agg3 = _sc_aggregate(16, xT3, zeros_init, src16, sidx16)
    fcwE = jnp.pad(fcW, ((0, 0), (0, 127)))
    batch3 = jnp.concatenate(
        [batch.astype(jnp.int32), jnp.full((NPAD - N,), 64, jnp.int32)]
    ).reshape(NPAD // BN, 1, BN)
    pooled2 = _tc_layer3_pool(h2, _agg_to_dense(agg3, 16, 256), cnt2,
                              W3, root3, b3.reshape(1, 32), fcwE, batch3)

    fin = _tc_finish(pooled2, fcb.reshape(1, 1))
    return fin[:, :1]


# speedup vs baseline: 9.9127x; 1.3270x over previous
import functools

import jax
import jax.numpy as jnp
from jax import lax
from jax.experimental import pallas as pl
from jax.experimental.pallas import tpu as pltpu
from jax.experimental.pallas import tpu_sc as plsc

N = 10000
E = 320000
R = 8
EPAD = 327680
SUB = 128
NSUB = 160
GRP = 8
ACC_ROWS = 80128
ZROWS = 313
IGRP = 32
DUMP = R * N
BN = 512
NPAD = 10240


def _sc_aggregate(k, xT, zeros_init, src16, sidx16):
    k_half = k // 2
    mesh = plsc.VectorSubcoreMesh(core_axis_name="c", subcore_axis_name="s")

    @functools.partial(
        pl.kernel,
        mesh=mesh,
        out_type=jax.ShapeDtypeStruct((R * N, k, 16), jnp.float32),
        compiler_params=pltpu.CompilerParams(use_tc_tiling_on_sc=False),
        scratch_types=[
            pltpu.VMEM((IGRP, SUB), jnp.int32),
            pltpu.VMEM((IGRP, SUB), jnp.int32),
            pltpu.VMEM((2 * GRP * SUB, 16), jnp.float32),
            pltpu.VMEM((ZROWS, 16), jnp.float32),
            pltpu.VMEM_SHARED((ACC_ROWS, 16), jnp.float32),
            pltpu.SemaphoreType.DMA,
            pltpu.SemaphoreType.DMA,
        ],
    )
    def body(xT_hbm, zer_hbm, src_hbm, sidx_hbm, out_hbm,
             src_t, sidx_t, rows_t, zbuf, acc, gsem, ssem):
        core = lax.axis_index("c")
        s = lax.axis_index("s")
        pltpu.sync_copy(zer_hbm, zbuf)
        ngrp = IGRP // GRP

        def fire_gathers(c, g, buf):
            return [pltpu.async_copy(
                xT_hbm.at[c].at[src_t.at[g * GRP + j]],
                rows_t.at[pl.ds(buf * GRP * SUB + j * SUB, SUB)], gsem)
                for j in range(GRP)]

        def chunk_body(cc, carry):
            c = core * k_half + cc
            for z in range(16):
                pltpu.sync_copy(zbuf, acc.at[pl.ds(s * 5008 + z * ZROWS, ZROWS)])
            plsc.subcore_barrier()

            def igrp_body(m, carry1):
                pltpu.sync_copy(src_hbm.at[s].at[pl.ds(m * IGRP, IGRP)], src_t)
                pltpu.sync_copy(sidx_hbm.at[s].at[pl.ds(m * IGRP, IGRP)],
                                sidx_t)
                gh = fire_gathers(c, 0, 0)
                sh = []
                for g in range(ngrp):
                    buf = g % 2
                    for h in gh:
                        h.wait()
                    for h in sh:
                        h.wait()
                    if g + 1 < ngrp:
                        gh = fire_gathers(c, g + 1, 1 - buf)
                    sh = [pltpu.async_copy(
                        rows_t.at[pl.ds(buf * GRP * SUB + j * SUB, SUB)],
                        acc.at[sidx_t.at[g * GRP + j]], ssem, add=True)
                        for j in range(GRP)]
                for h in sh:
                    h.wait()
                return carry1

            lax.fori_loop(0, NSUB // IGRP, igrp_body, 0)
            plsc.subcore_barrier()
            pltpu.sync_copy(acc.at[pl.ds(s * 5000, 5000)],
                            out_hbm.at[pl.ds(s * 5000, 5000), c])
            plsc.subcore_barrier()
            return carry

        lax.fori_loop(0, k_half, chunk_body, 0)

    return body(xT, zeros_init, src16, sidx16)


def _sc_count(sidx32, zeros_init, ones_init):
    mesh = plsc.VectorSubcoreMesh(core_axis_name="c", subcore_axis_name="s")

    @functools.partial(
        pl.kernel,
        mesh=mesh,
        out_type=jax.ShapeDtypeStruct((2, R * N, 16), jnp.float32),
        compiler_params=pltpu.CompilerParams(use_tc_tiling_on_sc=False),
        scratch_types=[
            pltpu.VMEM((80, SUB), jnp.int32),
            pltpu.VMEM((SUB, 16), jnp.float32),
            pltpu.VMEM((ZROWS, 16), jnp.float32),
            pltpu.VMEM_SHARED((ACC_ROWS, 16), jnp.float32),
        ],
    )
    def body(sidx_hbm, zer_hbm, one_hbm, out_hbm, sidx_t, ones_t, zbuf, acc):
        core = lax.axis_index("c")
        s = lax.axis_index("s")
        wid = s * 2 + core
        pltpu.sync_copy(sidx_hbm.at[wid], sidx_t)
        pltpu.sync_copy(one_hbm, ones_t)
        pltpu.sync_copy(zer_hbm, zbuf)
        for z in range(16):
            pltpu.sync_copy(zbuf, acc.at[pl.ds(s * 5008 + z * ZROWS, ZROWS)])
        plsc.subcore_barrier()

        def sub_body(g, carry):
            pltpu.sync_copy(ones_t, acc.at[sidx_t.at[g]], add=True)
            return carry

        lax.fori_loop(0, 80, sub_body, 0)
        plsc.subcore_barrier()
        pltpu.sync_copy(acc.at[pl.ds(s * 5000, 5000)],
                        out_hbm.at[core].at[pl.ds(s * 5000, 5000)])

    return body(sidx32, zeros_init, ones_init)


def _tc_layer(x, agg, cnt2, w, root, b, dout):
    d = x.shape[1]

    def body(x_ref, agg_ref, cnt_ref, w_ref, root_ref, b_ref, out_ref):
        xb = x_ref[...]
        cnt = cnt_ref[0] + cnt_ref[1]
        inv = 1.0 / jnp.maximum(cnt, 1.0)
        h = jnp.dot(xb, root_ref[...], preferred_element_type=jnp.float32)
        h = h + b_ref[...]
        for r in range(R):
            h = h + jnp.dot(agg_ref[r] * inv[r][:, None], w_ref[r],
                            preferred_element_type=jnp.float32)
        out_ref[...] = jnp.maximum(h, 0.0)

    return pl.pallas_call(
        body,
        grid=(NPAD // BN,),
        in_specs=[
            pl.BlockSpec((BN, d), lambda i: (i, 0)),
            pl.BlockSpec((R, BN, d), lambda i: (0, i, 0)),
            pl.BlockSpec((2, R, BN), lambda i: (0, 0, i)),
            pl.BlockSpec((R, d, dout), lambda i: (0, 0, 0)),
            pl.BlockSpec((d, dout), lambda i: (0, 0)),
            pl.BlockSpec((1, dout), lambda i: (0, 0)),
        ],
        out_specs=pl.BlockSpec((BN, dout), lambda i: (i, 0)),
        out_shape=jax.ShapeDtypeStruct((NPAD, dout), jnp.float32),
    )(x, agg, cnt2, w, root, b)


def _tc_layer3_pool(x, agg, cnt2, w, root, b, fcwE, batch3):
    d = x.shape[1]

    def body(x_ref, agg_ref, cnt_ref, w_ref, root_ref, b_ref, fcw_ref,
             bat_ref, out_ref):
        i = pl.program_id(0)
        xb = x_ref[...]
        cnt = cnt_ref[0] + cnt_ref[1]
        inv = 1.0 / jnp.maximum(cnt, 1.0)
        h = jnp.dot(xb, root_ref[...], preferred_element_type=jnp.float32)
        h = h + b_ref[...]
        for r in range(R):
            h = h + jnp.dot(agg_ref[r] * inv[r][:, None], w_ref[r],
                            preferred_element_type=jnp.float32)
        h = jnp.maximum(h, 0.0)
        zc = jnp.dot(h, fcw_ref[...], preferred_element_type=jnp.float32)
        ones_col = (lax.broadcasted_iota(jnp.int32, (BN, 128), 1) == 1)
        zc = zc + ones_col.astype(jnp.float32)
        bat = bat_ref[...].reshape(BN)
        oh = (bat[None, :] == lax.broadcasted_iota(jnp.int32, (64, BN), 0))
        contrib = jnp.dot(oh.astype(jnp.float32), zc,
                          preferred_element_type=jnp.float32)

        @pl.when(i == 0)
        def _():
            out_ref[...] = contrib

        @pl.when(i > 0)
        def _():
            out_ref[...] = out_ref[...] + contrib

    return pl.pallas_call(
        body,
        grid=(NPAD // BN,),
        in_specs=[
            pl.BlockSpec((BN, d), lambda i: (i, 0)),
            pl.BlockSpec((R, BN, d), lambda i: (0, i, 0)),
            pl.BlockSpec((2, R, BN), lambda i: (0, 0, i)),
            pl.BlockSpec((R, d, 32), lambda i: (0, 0, 0)),
            pl.BlockSpec((d, 32), lambda i: (0, 0)),
            pl.BlockSpec((1, 32), lambda i: (0, 0)),
            pl.BlockSpec((32, 128), lambda i: (0, 0)),
            pl.BlockSpec((1, 1, BN), lambda i: (i, 0, 0)),
        ],
        out_specs=pl.BlockSpec((64, 128), lambda i: (0, 0)),
        out_shape=jax.ShapeDtypeStruct((64, 128), jnp.float32),
    )(x, agg, cnt2, w, root, b, fcwE, batch3)


def _tc_finish(pooled2, fcb2):
    def body(p_ref, fcb_ref, out_ref):
        a = p_ref[...]
        v = a[:, 0:1] / jnp.maximum(a[:, 1:2], 1.0) + fcb_ref[0, 0]
        out_ref[...] = jnp.broadcast_to(v, (64, 128))

    return pl.pallas_call(
        body,
        in_specs=[pl.BlockSpec((64, 128), lambda: (0, 0)),
                  pl.BlockSpec((1, 1), lambda: (0, 0))],
        out_specs=pl.BlockSpec((64, 128), lambda: (0, 0)),
        out_shape=jax.ShapeDtypeStruct((64, 128), jnp.float32),
    )(pooled2, fcb2)


def _pad_nodes(a):
    return jnp.pad(a, ((0, NPAD - N), (0, 0)))


def _agg_to_dense(raw, k, d):
    t = raw.reshape(R, N, d)
    return jnp.pad(t, ((0, 0), (0, NPAD - N), (0, 0)))


def kernel(x, edge_index, edge_type, batch, W1, root1, b1, W2, root2, b2,
           W3, root3, b3, fcW, fcb):
    src = edge_index[0].astype(jnp.int32)
    dst = edge_index[1].astype(jnp.int32)
    et = edge_type.astype(jnp.int32)

    pad = EPAD - E
    src_p = jnp.concatenate([src, jnp.zeros((pad,), jnp.int32)])
    sidx = et * N + dst
    sidx_p = jnp.concatenate([sidx, jnp.full((pad,), DUMP, jnp.int32)])
    src16 = src_p.reshape(16, NSUB, SUB)
    sidx16 = sidx_p.reshape(16, NSUB, SUB)
    sidx32 = sidx_p.reshape(32, 80, SUB)
    zeros_init = jnp.zeros((ZROWS, 16), jnp.float32)
    ones_init = jnp.ones((SUB, 16), jnp.float32)

    cnt_raw = _sc_count(sidx32, zeros_init, ones_init)
    cnt2 = cnt_raw[:, :, 0].reshape(2, R, N)
    cnt2 = jnp.pad(cnt2, ((0, 0), (0, 0), (0, NPAD - N)))

    xT1 = x.reshape(N, 8, 16).transpose(1, 0, 2)
    agg1 = _sc_aggregate(8, xT1, zeros_init, src16, sidx16)
    h1 = _tc_layer(_pad_nodes(x), _agg_to_dense(agg1, 8, 128), cnt2,
                   W1, root1, b1.reshape(1, 256), 256)

    xT2 = h1[:N].reshape(N, 16, 16).transpose(1, 0, 2)
    agg2 = _sc_aggregate(16, xT2, zeros_init, src16, sidx16)
    h2 = _tc_layer(h1, _agg_to_dense(agg2, 16, 256), cnt2,
                   W2, root2, b2.reshape(1, 256), 256)

    xT3 = h2[:N].reshape(N, 16, 16).transpose(1, 0, 2)
    agg3 = _sc_aggregate(16, xT3, zeros_init, src16, sidx16)
    fcwE = jnp.pad(fcW, ((0, 0), (0, 127)))
    batch3 = jnp.concatenate(
        [batch.astype(jnp.int32), jnp.full((NPAD - N,), 64, jnp.int32)]
    ).reshape(NPAD // BN, 1, BN)
    pooled2 = _tc_layer3_pool(h2, _agg_to_dense(agg3, 16, 256), cnt2,
                              W3, root3, b3.reshape(1, 32), fcwE, batch3)

    fin = _tc_finish(pooled2, fcb.reshape(1, 1))
    return fin[:, :1]
